# Initial kernel scaffold; baseline (speedup 1.0000x reference)
import jax
import jax.numpy as jnp
from jax.experimental import pallas as pl
from jax.experimental.pallas import tpu as pltpu

INPUT_DIM = 768
PROJ_DIM = 64
TOP_K = 2048


def _score_body(x_ref, wq_ref, bq_ref, wk_ref, bk_ref, s_ref):
    x = x_ref[0]
    q = jax.lax.dot_general(x, wq_ref[...], (((1,), (1,)), ((), ())))
    q = q + bq_ref[...][None, :]
    k = jax.lax.dot_general(x, wk_ref[...], (((1,), (1,)), ((), ())))
    k = k + bk_ref[...][None, :]
    s = jax.lax.dot_general(q, k, (((1,), (1,)), ((), ())))
    s_ref[0] = s * (PROJ_DIM ** -0.5)


def _scores(x, Wq, bq, Wk, bk):
    B, L, D = x.shape
    return pl.pallas_call(
        _score_body,
        grid=(B,),
        in_specs=[
            pl.BlockSpec((1, L, D), lambda b: (b, 0, 0)),
            pl.BlockSpec((PROJ_DIM, D), lambda b: (0, 0)),
            pl.BlockSpec((PROJ_DIM,), lambda b: (0,)),
            pl.BlockSpec((PROJ_DIM, D), lambda b: (0, 0)),
            pl.BlockSpec((PROJ_DIM,), lambda b: (0,)),
        ],
        out_specs=pl.BlockSpec((1, L, L), lambda b: (b, 0, 0)),
        out_shape=jax.ShapeDtypeStruct((B, L, L), jnp.float32),
    )(x, Wq, bq, Wk, bk)


def kernel(x, padding_mask, Wq, bq, Wk, bk):
    B, L, _ = x.shape
    s = _scores(x, Wq, bq, Wk, bk)
    flat = s.reshape(B, L * L)
    k = min(TOP_K, L * L)
    topk_vals, topk_idx = jax.lax.top_k(flat, k)
    topk_weights = jax.nn.softmax(topk_vals, axis=-1)
    row_idx = topk_idx // L
    col_idx = topk_idx % L
    topk_indices = jnp.stack([row_idx, col_idx], axis=-1)
    return (topk_indices, topk_weights)


# speedup vs baseline: 1.0015x; 1.0015x over previous
import jax
import jax.numpy as jnp
from jax.experimental import pallas as pl
from jax.experimental.pallas import tpu as pltpu

INPUT_DIM = 768
PROJ_DIM = 64
TOP_K = 2048


def _score_body(x_ref, wq_ref, bq_ref, wk_ref, bk_ref, s_ref):
    x = x_ref[0]
    dn = (((1,), (1,)), ((), ()))
    q = jax.lax.dot_general(wq_ref[...], x, dn).T + bq_ref[...][None, :]
    k = jax.lax.dot_general(wk_ref[...], x, dn).T + bk_ref[...][None, :]
    s = jax.lax.dot_general(q, k, dn)
    s_ref[0] = s * (PROJ_DIM ** -0.5)


def _scores(x, Wq, bq, Wk, bk):
    B, L, D = x.shape
    return pl.pallas_call(
        _score_body,
        grid=(B,),
        in_specs=[
            pl.BlockSpec((1, L, D), lambda b: (b, 0, 0)),
            pl.BlockSpec((PROJ_DIM, D), lambda b: (0, 0)),
            pl.BlockSpec((PROJ_DIM,), lambda b: (0,)),
            pl.BlockSpec((PROJ_DIM, D), lambda b: (0, 0)),
            pl.BlockSpec((PROJ_DIM,), lambda b: (0,)),
        ],
        out_specs=pl.BlockSpec((1, L, L), lambda b: (b, 0, 0)),
        out_shape=jax.ShapeDtypeStruct((B, L, L), jnp.float32),
    )(x, Wq, bq, Wk, bk)


def kernel(x, padding_mask, Wq, bq, Wk, bk):
    B, L, _ = x.shape
    s = _scores(x, Wq, bq, Wk, bk)
    flat = s.reshape(B, L * L)
    k = min(TOP_K, L * L)
    topk_vals, topk_idx = jax.lax.top_k(flat, k)
    topk_weights = jax.nn.softmax(topk_vals, axis=-1)
    row_idx = topk_idx // L
    col_idx = topk_idx % L
    topk_indices = jnp.stack([row_idx, col_idx], axis=-1)
    return (topk_indices, topk_weights)


# speedup vs baseline: 3.3604x; 3.3552x over previous
import jax
import jax.numpy as jnp
from jax.experimental import pallas as pl
from jax.experimental.pallas import tpu as pltpu

INPUT_DIM = 768
PROJ_DIM = 64
TOP_K = 2048
L = 2048
NCHUNK = 16
KEEP_CHUNKS = 2560


def _score_body(x_ref, wq_ref, bq_ref, wk_ref, bk_ref,
                s_hbm, cmax_ref, s_ref, sem):
    b = pl.program_id(0)
    x = x_ref[0]
    dn = (((1,), (1,)), ((), ()))
    q = jax.lax.dot_general(wq_ref[...], x, dn).T + bq_ref[...][None, :]
    k = jax.lax.dot_general(wk_ref[...], x, dn).T + bk_ref[...][None, :]
    s_ref[...] = jax.lax.dot_general(q, k, dn) * (PROJ_DIM ** -0.5)
    cpy = pltpu.make_async_copy(s_ref, s_hbm.at[b], sem)
    cpy.start()

    cm_parts = []
    for i in range(8):
        blk = s_ref[pl.ds(i * 256, 256), :]
        cm_parts.append(jnp.max(blk.reshape(256, NCHUNK, 128), axis=2))
    cm = jnp.concatenate(cm_parts, axis=0)
    cmax_ref[0] = jnp.pad(cm, ((0, 0), (0, 128 - NCHUNK)),
                          constant_values=-jnp.inf)
    cpy.wait()


def _scores_and_chunkmax(x, Wq, bq, Wk, bk):
    B = x.shape[0]
    return pl.pallas_call(
        _score_body,
        grid=(B,),
        in_specs=[
            pl.BlockSpec((1, L, INPUT_DIM), lambda b: (b, 0, 0)),
            pl.BlockSpec((PROJ_DIM, INPUT_DIM), lambda b: (0, 0)),
            pl.BlockSpec((PROJ_DIM,), lambda b: (0,)),
            pl.BlockSpec((PROJ_DIM, INPUT_DIM), lambda b: (0, 0)),
            pl.BlockSpec((PROJ_DIM,), lambda b: (0,)),
        ],
        out_specs=[
            pl.BlockSpec(memory_space=pltpu.MemorySpace.HBM),
            pl.BlockSpec((1, L, 128), lambda b: (b, 0, 0)),
        ],
        scratch_shapes=[pltpu.VMEM((L, L), jnp.float32),
                        pltpu.SemaphoreType.DMA],
        out_shape=[
            jax.ShapeDtypeStruct((B, L, L), jnp.float32),
            jax.ShapeDtypeStruct((B, L, 128), jnp.float32),
        ],
    )(x, Wq, bq, Wk, bk)


def kernel(x, padding_mask, Wq, bq, Wk, bk):
    B = x.shape[0]
    s, cmax_pad = _scores_and_chunkmax(x, Wq, bq, Wk, bk)
    cmax = cmax_pad[:, :, :NCHUNK].reshape(B, L * NCHUNK)
    _, chunk_ids = jax.lax.top_k(cmax, KEEP_CHUNKS)
    s3 = s.reshape(B, L * NCHUNK, 128)
    cand = jnp.take_along_axis(s3, chunk_ids[:, :, None], axis=1)
    flat = cand.reshape(B, KEEP_CHUNKS * 128)
    topk_vals, pos = jax.lax.top_k(flat, TOP_K)
    flat_idx = (jnp.take_along_axis(chunk_ids, pos // 128, axis=1) * 128
                + pos % 128)
    _, _, topk_vals, flat_idx = jax.lax.sort(
        (-topk_vals, flat_idx, topk_vals, flat_idx), dimension=1, num_keys=2)
    topk_weights = jax.nn.softmax(topk_vals, axis=-1)
    row_idx = flat_idx // L
    col_idx = flat_idx % L
    topk_indices = jnp.stack([row_idx, col_idx], axis=-1)
    return (topk_indices, topk_weights)


# speedup vs baseline: 3.3705x; 1.0030x over previous
import jax
import jax.numpy as jnp
from jax.experimental import pallas as pl
from jax.experimental.pallas import tpu as pltpu

INPUT_DIM = 768
PROJ_DIM = 64
TOP_K = 2048
L = 2048
NCHUNK = 16
KEEP_CHUNKS = 2176


def _score_body(x_ref, wq_ref, bq_ref, wk_ref, bk_ref,
                s_hbm, cmax_ref, s_ref, sem):
    b = pl.program_id(0)
    x = x_ref[0]
    dn = (((1,), (1,)), ((), ()))
    q = jax.lax.dot_general(wq_ref[...], x, dn).T + bq_ref[...][None, :]
    k = jax.lax.dot_general(wk_ref[...], x, dn).T + bk_ref[...][None, :]
    s_ref[...] = jax.lax.dot_general(q, k, dn) * (PROJ_DIM ** -0.5)
    cpy = pltpu.make_async_copy(s_ref, s_hbm.at[b], sem)
    cpy.start()

    cm_parts = []
    for i in range(8):
        blk = s_ref[pl.ds(i * 256, 256), :]
        cm_parts.append(jnp.max(blk.reshape(256, NCHUNK, 128), axis=2))
    cm = jnp.concatenate(cm_parts, axis=0)
    cmax_ref[0] = jnp.pad(cm, ((0, 0), (0, 128 - NCHUNK)),
                          constant_values=-jnp.inf)
    cpy.wait()


def _scores_and_chunkmax(x, Wq, bq, Wk, bk):
    B = x.shape[0]
    return pl.pallas_call(
        _score_body,
        grid=(B,),
        in_specs=[
            pl.BlockSpec((1, L, INPUT_DIM), lambda b: (b, 0, 0)),
            pl.BlockSpec((PROJ_DIM, INPUT_DIM), lambda b: (0, 0)),
            pl.BlockSpec((PROJ_DIM,), lambda b: (0,)),
            pl.BlockSpec((PROJ_DIM, INPUT_DIM), lambda b: (0, 0)),
            pl.BlockSpec((PROJ_DIM,), lambda b: (0,)),
        ],
        out_specs=[
            pl.BlockSpec(memory_space=pltpu.MemorySpace.HBM),
            pl.BlockSpec((1, L, 128), lambda b: (b, 0, 0)),
        ],
        scratch_shapes=[pltpu.VMEM((L, L), jnp.float32),
                        pltpu.SemaphoreType.DMA],
        out_shape=[
            jax.ShapeDtypeStruct((B, L, L), jnp.float32),
            jax.ShapeDtypeStruct((B, L, 128), jnp.float32),
        ],
    )(x, Wq, bq, Wk, bk)


def kernel(x, padding_mask, Wq, bq, Wk, bk):
    B = x.shape[0]
    s, cmax_pad = _scores_and_chunkmax(x, Wq, bq, Wk, bk)
    cmax = cmax_pad[:, :, :NCHUNK].reshape(B, L * NCHUNK)
    _, chunk_ids = jax.lax.top_k(cmax, KEEP_CHUNKS)
    s3 = s.reshape(B, L * NCHUNK, 128)
    cand = jnp.take_along_axis(s3, chunk_ids[:, :, None], axis=1)
    flat = cand.reshape(B, KEEP_CHUNKS * 128)
    topk_vals, pos = jax.lax.top_k(flat, TOP_K)
    flat_idx = (jnp.take_along_axis(chunk_ids, pos // 128, axis=1) * 128
                + pos % 128)
    _, _, topk_vals, flat_idx = jax.lax.sort(
        (-topk_vals, flat_idx, topk_vals, flat_idx), dimension=1, num_keys=2)
    topk_weights = jax.nn.softmax(topk_vals, axis=-1)
    row_idx = flat_idx // L
    col_idx = flat_idx % L
    topk_indices = jnp.stack([row_idx, col_idx], axis=-1)
    return (topk_indices, topk_weights)
